# Initial kernel scaffold; baseline (speedup 1.0000x reference)
import jax
import jax.numpy as jnp
from jax.experimental import pallas as pl


def kernel(vis, pos, grad, lib, B_f, pos_w, pos_b, img_w, img_b, router_w, router_b, ew1, eb1, ew2, eb2, dec_w1, dec_b1, ln_g, ln_b, dec_w2, dec_b2, al_w1, al_b1, al_w2, al_b2, fn_w1, fn_b1, fn_w2, fn_b2):
    raise NotImplementedError("write your pallas kernel here")


# speedup vs baseline: 2.6411x; 2.6411x over previous
import functools

import jax
import jax.numpy as jnp
import numpy as np
from jax.experimental import pallas as pl
from jax.experimental.pallas import tpu as pltpu

NUM_GENES = 2000
DH = 256
DU = 1024
NE = 4
BATCH = 8192
BB = 512
NB = BATCH // BB


def _gelu(x):
    return 0.5 * x * (1.0 + jax.lax.erf(x * np.float32(1.0 / np.sqrt(2.0))))


def _enc_body(vis, pos, grad, B_f, pos_w, pos_b, img_w, img_b, rw, rb,
              z_out, eid_out, gate_out):
    xp = jnp.float32(2.0 * np.pi) * jnp.dot(pos[...], B_f[...],
                                            preferred_element_type=jnp.float32)
    four = jnp.concatenate([jnp.sin(xp), jnp.cos(xp)], axis=-1)
    penc = _gelu(jnp.dot(four, pos_w[...], preferred_element_type=jnp.float32)
                 + pos_b[...])
    z = jnp.dot(vis[...], img_w[...], preferred_element_type=jnp.float32) \
        + img_b[...] + penc
    z_out[...] = z
    rw_z = rw[0:DH, :]
    rw_g = rw[DH:DH + 1, :]
    logits = jnp.dot(z, rw_z, preferred_element_type=jnp.float32) \
        + grad[...] * rw_g + rb[...]
    m = jnp.max(logits, axis=-1, keepdims=True)
    e = jnp.exp(logits - m)
    probs = e / jnp.sum(e, axis=-1, keepdims=True)
    pmax = jnp.max(probs, axis=-1, keepdims=True)
    ids = jax.lax.broadcasted_iota(jnp.int32, probs.shape, 1)
    eid = jnp.min(jnp.where(probs >= pmax, ids, NE), axis=-1, keepdims=True)
    eid_out[...] = eid
    gate_out[...] = pmax


def _encoder(vis, pos, grad, B_f, pos_w, pos_b, img_w, img_b, rw, rb):
    return pl.pallas_call(
        _enc_body,
        grid=(NB,),
        in_specs=[
            pl.BlockSpec((BB, DU), lambda i: (i, 0)),
            pl.BlockSpec((BB, 3), lambda i: (i, 0)),
            pl.BlockSpec((BB, 1), lambda i: (i, 0)),
            pl.BlockSpec((3, 128), lambda i: (0, 0)),
            pl.BlockSpec((DH, DH), lambda i: (0, 0)),
            pl.BlockSpec((DH,), lambda i: (0,)),
            pl.BlockSpec((DU, DH), lambda i: (0, 0)),
            pl.BlockSpec((DH,), lambda i: (0,)),
            pl.BlockSpec((DH + 1, NE), lambda i: (0, 0)),
            pl.BlockSpec((NE,), lambda i: (0,)),
        ],
        out_specs=[
            pl.BlockSpec((BB, DH), lambda i: (i, 0)),
            pl.BlockSpec((BB, 1), lambda i: (i, 0)),
            pl.BlockSpec((BB, 1), lambda i: (i, 0)),
        ],
        out_shape=[
            jax.ShapeDtypeStruct((BATCH, DH), jnp.float32),
            jax.ShapeDtypeStruct((BATCH, 1), jnp.int32),
            jax.ShapeDtypeStruct((BATCH, 1), jnp.float32),
        ],
        compiler_params=pltpu.CompilerParams(
            dimension_semantics=("arbitrary",)),
    )(vis, pos, grad, B_f, pos_w, pos_b, img_w, img_b, rw, rb)


def _moe_body(z, eid, ew1, eb1, ew2, eb2, out):
    zb = z[...]
    acc = jnp.zeros_like(zb)
    for e in range(NE):
        h = _gelu(jnp.dot(zb, ew1[e], preferred_element_type=jnp.float32)
                  + eb1[e])
        h = jnp.dot(h, ew2[e], preferred_element_type=jnp.float32) + eb2[e]
        acc = acc + jnp.where(eid[...] == e, h, 0.0)
    out[...] = acc


def _moe_dense(z, eid, ew1, eb1, ew2, eb2):
    return pl.pallas_call(
        _moe_body,
        grid=(NB,),
        in_specs=[
            pl.BlockSpec((BB, DH), lambda i: (i, 0)),
            pl.BlockSpec((BB, 1), lambda i: (i, 0)),
            pl.BlockSpec((NE, DH, 4 * DH), lambda i: (0, 0, 0)),
            pl.BlockSpec((NE, 4 * DH), lambda i: (0, 0)),
            pl.BlockSpec((NE, 4 * DH, DH), lambda i: (0, 0, 0)),
            pl.BlockSpec((NE, DH), lambda i: (0, 0)),
        ],
        out_specs=pl.BlockSpec((BB, DH), lambda i: (i, 0)),
        out_shape=jax.ShapeDtypeStruct((BATCH, DH), jnp.float32),
        compiler_params=pltpu.CompilerParams(
            dimension_semantics=("arbitrary",)),
    )(z, eid, ew1, eb1, ew2, eb2)


def _dec_body(z, moe, gate, lib, dw1, db1, ln_g, ln_b, w_mu, b_mu, w_th, b_th,
              aw1, ab1, aw2, ab2, fw1, fb1, fw2, fb2,
              mu_out, th_out, fn_out, al_out):
    z2 = z[...] + gate[...] * moe[...]
    h = jnp.dot(z2, dw1[...], preferred_element_type=jnp.float32) + db1[...]
    m = jnp.mean(h, axis=-1, keepdims=True)
    hc = h - m
    v = jnp.mean(hc * hc, axis=-1, keepdims=True)
    h = hc * jax.lax.rsqrt(v + 1e-5) * ln_g[...] + ln_b[...]
    h = _gelu(h)
    mu_lin = jnp.dot(h, w_mu[...], preferred_element_type=jnp.float32) + b_mu[...]
    th_lin = jnp.dot(h, w_th[...], preferred_element_type=jnp.float32) + b_th[...]
    sp = lambda x: jnp.logaddexp(x, 0.0)
    mu_out[...] = sp(mu_lin) * lib[...] + 1e-06
    th_out[...] = sp(th_lin) + 1e-06
    fh = _gelu(jnp.dot(z2, fw1[...], preferred_element_type=jnp.float32) + fb1[...])
    fn = jnp.dot(fh, fw2[...], preferred_element_type=jnp.float32) + fb2[...]
    fn_out[...] = jax.nn.sigmoid(fn)
    ah = _gelu(jnp.dot(z2, aw1[...], preferred_element_type=jnp.float32) + ab1[...])
    al_out[...] = jnp.dot(ah, aw2[...], preferred_element_type=jnp.float32) + ab2[...]


def _decoder(z, moe, gate, lib, dw1, db1, ln_g, ln_b, w_mu, b_mu, w_th, b_th,
             aw1, ab1, aw2, ab2, fw1, fb1, fw2, fb2):
    full = lambda *shape: pl.BlockSpec(shape, lambda i: (0,) * len(shape))
    row = lambda *shape: pl.BlockSpec(shape, lambda i: (i,) + (0,) * (len(shape) - 1))
    return pl.pallas_call(
        _dec_body,
        grid=(NB,),
        in_specs=[
            row(BB, DH), row(BB, DH), row(BB, 1), row(BB, 1),
            full(DH, DH), full(DH), full(DH), full(DH),
            full(DH, NUM_GENES), full(NUM_GENES),
            full(DH, NUM_GENES), full(NUM_GENES),
            full(DH, 128), full(128), full(128, 30), full(30),
            full(DH, 64), full(64), full(64, 1), full(1),
        ],
        out_specs=[
            row(BB, NUM_GENES), row(BB, NUM_GENES), row(BB, 1), row(BB, 30),
        ],
        out_shape=[
            jax.ShapeDtypeStruct((BATCH, NUM_GENES), jnp.float32),
            jax.ShapeDtypeStruct((BATCH, NUM_GENES), jnp.float32),
            jax.ShapeDtypeStruct((BATCH, 1), jnp.float32),
            jax.ShapeDtypeStruct((BATCH, 30), jnp.float32),
        ],
        compiler_params=pltpu.CompilerParams(
            dimension_semantics=("arbitrary",)),
    )(z, moe, gate, lib, dw1, db1, ln_g, ln_b, w_mu, b_mu, w_th, b_th,
      aw1, ab1, aw2, ab2, fw1, fb1, fw2, fb2)


def kernel(vis, pos, grad, lib, B_f, pos_w, pos_b, img_w, img_b, router_w,
           router_b, ew1, eb1, ew2, eb2, dec_w1, dec_b1, ln_g, ln_b, dec_w2,
           dec_b2, al_w1, al_b1, al_w2, al_b2, fn_w1, fn_b1, fn_w2, fn_b2):
    z, eid, gate = _encoder(vis, pos, grad, B_f, pos_w, pos_b, img_w, img_b,
                            router_w, router_b)
    moe = _moe_dense(z, eid, ew1, eb1, ew2, eb2)
    w_mu = dec_w2[:, 0::2]
    w_th = dec_w2[:, 1::2]
    b_mu = dec_b2[0::2]
    b_th = dec_b2[1::2]
    mu, theta, func, align = _decoder(
        z, moe, gate, lib, dec_w1, dec_b1, ln_g, ln_b, w_mu, b_mu, w_th, b_th,
        al_w1, al_b1, al_w2, al_b2, fn_w1, fn_b1, fn_w2, fn_b2)
    return (mu, theta, func, align)
